# baseline (speedup 1.0000x reference)
import dataclasses
import functools

import jax
import jax.numpy as jnp
from jax import lax
from jax.experimental import pallas as pl
from jax.experimental.pallas import tpu as pltpu
from jax.experimental.pallas import tpu_sc as plsc

NC = 2
NS = 16
L = 16
NW = NC * NS
REP = 8


@functools.lru_cache(maxsize=None)
def _bag_call(N, T, V, D):
    rows_w = N // NW
    n_groups = rows_w // L
    pairs = D // 2
    vpad = -(-V // 16) * 16
    sub = vpad * REP

    mesh = plsc.VectorSubcoreMesh(core_axis_name="c", subcore_axis_name="s")
    cp = pltpu.CompilerParams()
    if "needs_layout_passes" in pltpu.CompilerParams.__dataclass_fields__:
        cp = dataclasses.replace(cp, needs_layout_passes=False)
    if "use_tc_tiling_on_sc" in pltpu.CompilerParams.__dataclass_fields__:
        cp = dataclasses.replace(cp, use_tc_tiling_on_sc=True)

    @functools.partial(
        pl.kernel,
        out_type=jax.ShapeDtypeStruct((NW, D, rows_w), jnp.float32),
        mesh=mesh,
        compiler_params=cp,
        scratch_types=[
            pltpu.VMEM((T, rows_w), jnp.int32),
            pltpu.VMEM((rows_w,), jnp.int32),
            pltpu.VMEM((pairs * sub,), jnp.int32),
            pltpu.VMEM((D, rows_w), jnp.float32),
            pltpu.VMEM((n_groups * T * L,), jnp.int32),
            pltpu.VMEM((rows_w,), jnp.float32),
            pltpu.SemaphoreType.DMA,
        ],
    )
    def bag(charsT_hbm, len_hbm, table_hbm, out_hbm,
            charsT_v, len_v, table_v, outT_v, cms_v, invl_v, tsem):
        wid = lax.axis_index("s") * NC + lax.axis_index("c")
        rbase = wid * rows_w
        tcopy = pltpu.make_async_copy(table_hbm, table_v, tsem)
        tcopy.start()
        pltpu.sync_copy(charsT_hbm.at[wid], charsT_v)
        pltpu.sync_copy(len_hbm.at[pl.ds(rbase, rows_w)], len_v)

        iota = lax.iota(jnp.int32, L)
        lane_rep = jnp.bitwise_and(iota, REP - 1)
        himask = jnp.full((L,), -65536, jnp.int32)

        @pl.loop(0, n_groups)
        def _(g):
            sl = pl.ds(g * L, L)
            lenv = len_v[sl]
            invl_v[sl] = 1.0 / jnp.maximum(lenv.astype(jnp.float32), 1.0)
            for t in range(T):
                c = charsT_v[t, sl]
                cm = jnp.where(t < lenv, c, jnp.zeros_like(c))
                cms_v[pl.ds((g * T + t) * L, L)] = cm * REP + lane_rep

        tcopy.wait()

        @pl.loop(0, n_groups)
        def _(g):
            sl = pl.ds(g * L, L)
            invl = invl_v[sl]
            cms = [cms_v[pl.ds((g * T + t) * L, L)] for t in range(T)]
            for p in range(pairs):
                tv = table_v.at[pl.ds(p * sub, sub)]
                ws = [plsc.load_gather(tv, [cms[t]]) for t in range(T)]
                sums = [
                    plsc.bitcast(
                        plsc.bitcast(ws[k], jnp.bfloat16)
                        + plsc.bitcast(ws[k + 1], jnp.bfloat16),
                        jnp.int32)
                    for k in range(0, T - 1, 2)
                ]
                if T % 2:
                    sums.append(ws[-1])
                los = [plsc.bitcast(jnp.left_shift(s, 16), jnp.float32)
                       for s in sums]
                his = [plsc.bitcast(jnp.bitwise_and(s, himask), jnp.float32)
                       for s in sums]

                def _tree(vs):
                    while len(vs) > 1:
                        nxt = [vs[i] + vs[i + 1]
                               for i in range(0, len(vs) - 1, 2)]
                        if len(vs) % 2:
                            nxt.append(vs[-1])
                        vs = nxt
                    return vs[0]

                outT_v[2 * p, sl] = _tree(los) * invl
                outT_v[2 * p + 1, sl] = _tree(his) * invl

        pltpu.sync_copy(outT_v, out_hbm.at[wid])

    return bag


@functools.lru_cache(maxsize=None)
def _proj_call(NB, D, R, H):
    def body(xt_ref, w_ref, o_ref):
        o_ref[...] = lax.dot_general(
            xt_ref[...], w_ref[...],
            dimension_numbers=(((1,), (0,)), ((), ())),
            preferred_element_type=jnp.float32,
        )

    return pl.pallas_call(
        body,
        in_specs=[
            pl.BlockSpec((NB, D, R), lambda: (0, 0, 0)),
            pl.BlockSpec((D, H), lambda: (0, 0)),
        ],
        out_specs=pl.BlockSpec((NB, R, H), lambda: (0, 0, 0)),
        out_shape=jax.ShapeDtypeStruct((NB, R, H), jnp.float32),
    )


def _pack_table(emb):
    V, D = emb.shape
    vpad = -(-V // 16) * 16
    pairs_bf = emb.astype(jnp.bfloat16).reshape(V, D // 2, 2)
    packed = lax.bitcast_convert_type(pairs_bf, jnp.uint32)
    packed = packed.T
    packed = jnp.pad(packed, ((0, 0), (0, vpad - V)))
    packed = jnp.repeat(packed[:, :, None], REP, axis=2)
    return lax.bitcast_convert_type(packed, jnp.int32).reshape(-1)


def kernel(char_input, lengths, emb, W):
    B, S, T = char_input.shape
    N = B * S
    V, D = emb.shape
    H = W.shape[1]
    rows_w = N // NW

    charsT = (char_input.reshape(NW, rows_w, T)
              .transpose(0, 2, 1)
              .astype(jnp.int32))
    flat_len = lengths.reshape(N)
    len_i32 = flat_len.astype(jnp.int32)

    pooledT = _bag_call(N, T, V, D)(charsT, len_i32, _pack_table(emb))
    out = _proj_call(NW, D, rows_w, H)(pooledT, W)
    return out.reshape(B, S, H), flat_len


# speedup vs baseline: 1.1074x; 1.1074x over previous
import dataclasses
import functools

import jax
import jax.numpy as jnp
from jax import lax
from jax.experimental import pallas as pl
from jax.experimental.pallas import tpu as pltpu
from jax.experimental.pallas import tpu_sc as plsc

NC = 2
NS = 16
L = 16
NW = NC * NS
HALVES = 2


@functools.lru_cache(maxsize=None)
def _bag_call(N, T, V, D):
    n_blocks = NW // HALVES
    rows_b = N // n_blocks
    n_groups = rows_b // L
    pairs_h = D // 2 // HALVES
    vpad = -(-V // 8) * 8
    sub = vpad * L

    mesh = plsc.VectorSubcoreMesh(core_axis_name="c", subcore_axis_name="s")
    cp = pltpu.CompilerParams()
    if "needs_layout_passes" in pltpu.CompilerParams.__dataclass_fields__:
        cp = dataclasses.replace(cp, needs_layout_passes=False)
    if "use_tc_tiling_on_sc" in pltpu.CompilerParams.__dataclass_fields__:
        cp = dataclasses.replace(cp, use_tc_tiling_on_sc=True)

    @functools.partial(
        pl.kernel,
        out_type=jax.ShapeDtypeStruct(
            (n_blocks, HALVES, D // HALVES, rows_b), jnp.float32),
        mesh=mesh,
        compiler_params=cp,
        scratch_types=[
            pltpu.VMEM((T, rows_b), jnp.int32),
            pltpu.VMEM((rows_b,), jnp.int32),
            pltpu.VMEM((pairs_h * sub,), jnp.int32),
            pltpu.VMEM((D // HALVES, rows_b), jnp.float32),
        ],
    )
    def bag(charsT_hbm, len_hbm, table_hbm, out_hbm,
            charsT_v, len_v, table_v, outT_v):
        wid = lax.axis_index("s") * NC + lax.axis_index("c")
        blk = wid // HALVES
        half = wid - blk * HALVES
        rbase = blk * rows_b
        pltpu.sync_copy(charsT_hbm.at[blk], charsT_v)
        pltpu.sync_copy(len_hbm.at[pl.ds(rbase, rows_b)], len_v)
        pltpu.sync_copy(table_hbm.at[half], table_v)

        iota = lax.iota(jnp.int32, L)
        himask = jnp.full((L,), -65536, jnp.int32)

        @pl.loop(0, n_groups)
        def _(g):
            sl = pl.ds(g * L, L)
            lenv = len_v[sl]
            invl = 1.0 / jnp.maximum(lenv.astype(jnp.float32), 1.0)
            cms = []
            for t in range(T):
                c = charsT_v[t, sl]
                cm = jnp.where(t < lenv, c, jnp.zeros_like(c))
                cms.append(cm * L + iota)
            for p in range(pairs_h):
                tv = table_v.at[pl.ds(p * sub, sub)]
                ws = [plsc.load_gather(tv, [cms[t]]) for t in range(T)]
                sums = [
                    plsc.bitcast(
                        plsc.bitcast(ws[k], jnp.bfloat16)
                        + plsc.bitcast(ws[k + 1], jnp.bfloat16),
                        jnp.int32)
                    for k in range(0, T - 1, 2)
                ]
                if T % 2:
                    sums.append(ws[-1])
                los = [plsc.bitcast(jnp.left_shift(s, 16), jnp.float32)
                       for s in sums]
                his = [plsc.bitcast(jnp.bitwise_and(s, himask), jnp.float32)
                       for s in sums]

                def _tree(vs):
                    while len(vs) > 1:
                        nxt = [vs[i] + vs[i + 1]
                               for i in range(0, len(vs) - 1, 2)]
                        if len(vs) % 2:
                            nxt.append(vs[-1])
                        vs = nxt
                    return vs[0]

                outT_v[2 * p, sl] = _tree(los) * invl
                outT_v[2 * p + 1, sl] = _tree(his) * invl

        pltpu.sync_copy(outT_v, out_hbm.at[blk, half])

    return bag


@functools.lru_cache(maxsize=None)
def _proj_call(NB, D, R, H):
    def body(xt_ref, w_ref, o_ref):
        o_ref[...] = lax.dot_general(
            xt_ref[...], w_ref[...],
            dimension_numbers=(((1,), (0,)), ((), ())),
            preferred_element_type=jnp.float32,
        )

    return pl.pallas_call(
        body,
        in_specs=[
            pl.BlockSpec((NB, D, R), lambda: (0, 0, 0)),
            pl.BlockSpec((D, H), lambda: (0, 0)),
        ],
        out_specs=pl.BlockSpec((NB, R, H), lambda: (0, 0, 0)),
        out_shape=jax.ShapeDtypeStruct((NB, R, H), jnp.float32),
    )


def _pack_table(emb):
    V, D = emb.shape
    vpad = -(-V // 8) * 8
    pairs_bf = emb.astype(jnp.bfloat16).reshape(V, D // 2, 2)
    packed = lax.bitcast_convert_type(pairs_bf, jnp.uint32)
    packed = packed.T
    packed = jnp.pad(packed, ((0, 0), (0, vpad - V)))
    packed = jnp.repeat(packed[:, :, None], L, axis=2)
    return lax.bitcast_convert_type(packed, jnp.int32).reshape(HALVES, -1)


def kernel(char_input, lengths, emb, W):
    B, S, T = char_input.shape
    N = B * S
    V, D = emb.shape
    H = W.shape[1]
    n_blocks = NW // HALVES
    rows_b = N // n_blocks

    charsT = (char_input.reshape(n_blocks, rows_b, T)
              .transpose(0, 2, 1)
              .astype(jnp.int32))
    flat_len = lengths.reshape(N)
    len_i32 = flat_len.astype(jnp.int32)

    pooledT = _bag_call(N, T, V, D)(charsT, len_i32, _pack_table(emb))
    out = _proj_call(n_blocks, D, rows_b, H)(
        pooledT.reshape(n_blocks, D, rows_b), W)
    return out.reshape(B, S, H), flat_len
